# baseline (speedup 1.0000x reference)
import functools

import jax
import jax.numpy as jnp
from jax.experimental import pallas as pl
from jax.experimental.pallas import tpu as pltpu

_VMEM_LIMIT = 48 * 1024 * 1024


def _round_up(x, m):
    return ((x + m - 1) // m) * m


def _conv_kernel(x_ref, w_ref, s_ref, b_ref, o_ref, *, th, wp, w_out, pool,
                 fatten):
    cout = o_ref.shape[-1]
    scale = s_ref[...]
    bias = b_ref[...]

    if not pool:
        cin = x_ref.shape[-1]
        xv = x_ref[0, 0]
        if fatten == "full9":
            xcat = jnp.concatenate(
                [xv[ky:ky + th, kx:kx + wp]
                 for ky in range(3) for kx in range(3)], axis=-1)
            acc = jnp.dot(xcat.reshape(th * wp, 9 * cin), w_ref[...],
                          preferred_element_type=jnp.float32)
        elif fatten == "ky3":
            xcat = jnp.concatenate([xv[ky:ky + th] for ky in range(3)],
                                   axis=-1)
            acc = None
            for kx in range(3):
                lhs = xcat[:, kx:kx + wp].reshape(th * wp, 3 * cin)
                d = jnp.dot(lhs, w_ref[kx],
                            preferred_element_type=jnp.float32)
                acc = d if acc is None else acc + d
        else:
            acc = None
            for ky in range(3):
                xr = xv[ky:ky + th]
                for kx in range(3):
                    lhs = xr[:, kx:kx + wp, :].reshape(th * wp, cin)
                    d = jnp.dot(lhs, w_ref[ky, kx],
                                preferred_element_type=jnp.float32)
                    acc = d if acc is None else acc + d
        y = jnp.maximum(acc * scale + bias, 0.0)
        o_ref[0] = y.reshape(th, wp, cout)[:, :w_out].astype(o_ref.dtype)
        return

    cin = x_ref.shape[-1] // 2
    wh = wp // 2
    wo = w_out // 2
    xf = x_ref[0, 0]
    xe = xf[:, :, :cin]
    xo = xf[:, :, cin:]

    def tap_dots(taps, wref_for_kx):
        acc = None
        for kx, tap in enumerate(taps):
            lhs = tap.reshape(th * wh, tap.shape[-1])
            d = jnp.dot(lhs, wref_for_kx(kx),
                        preferred_element_type=jnp.float32)
            acc = d if acc is None else acc + d
        return acc

    if fatten == "ky3":
        xec = jnp.concatenate([xe[ky:ky + th] for ky in range(3)], axis=-1)
        xoc = jnp.concatenate([xo[ky:ky + th] for ky in range(3)], axis=-1)
        acc_e = tap_dots([xec[:, 0:wh], xoc[:, 0:wh], xec[:, 1:1 + wh]],
                         lambda kx: w_ref[kx])
        acc_o = tap_dots([xoc[:, 0:wh], xec[:, 1:1 + wh], xoc[:, 1:1 + wh]],
                         lambda kx: w_ref[kx])
    else:
        acc_e = acc_o = None
        for ky in range(3):
            xer = xe[ky:ky + th]
            xor = xo[ky:ky + th]
            for kx, tap in enumerate((xer[:, 0:wh], xor[:, 0:wh],
                                      xer[:, 1:1 + wh])):
                d = jnp.dot(tap.reshape(th * wh, cin), w_ref[ky, kx],
                            preferred_element_type=jnp.float32)
                acc_e = d if acc_e is None else acc_e + d
            for kx, tap in enumerate((xor[:, 0:wh], xer[:, 1:1 + wh],
                                      xor[:, 1:1 + wh])):
                d = jnp.dot(tap.reshape(th * wh, cin), w_ref[ky, kx],
                            preferred_element_type=jnp.float32)
                acc_o = d if acc_o is None else acc_o + d

    y_e = jnp.maximum(acc_e * scale + bias, 0.0)
    y_o = jnp.maximum(acc_o * scale + bias, 0.0)
    y = jnp.maximum(y_e, y_o).reshape(th, wh, cout)[:, :wo]
    y = y.reshape(th // 2, 2, wo, cout).max(axis=1)
    o_ref[0] = y.astype(o_ref.dtype)


def _conv3x3(x, w, scale, bias, *, pool):
    B, H, W, Cin = x.shape
    Cout = w.shape[-1]

    if Cin % 8 != 0:
        pc = 8 - Cin % 8
        x = jnp.pad(x, ((0, 0), (0, 0), (0, 0), (0, pc)))
        w = jnp.pad(w, ((0, 0), (0, 0), (0, pc), (0, 0)))
        Cin += pc

    Wp = _round_up(W, 8)
    xp = jnp.pad(x, ((0, 0), (1, 1), (1, Wp - W + 1), (0, 0)))

    th = 56 if H % 56 == 0 else H
    nt = H // th
    if nt > 1:
        xs = jnp.stack([xp[:, t * th:t * th + th + 2] for t in range(nt)],
                       axis=1)
    else:
        xs = xp[:, None]
    tho, wo = (th // 2, W // 2) if pool else (th, W)

    if pool:
        xs = xs.reshape(B, nt, th + 2, (Wp + 2) // 2, 2 * Cin)
        x_spec = pl.BlockSpec((1, 1, th + 2, (Wp + 2) // 2, 2 * Cin),
                              lambda b, t: (b, t, 0, 0, 0))
    else:
        x_spec = pl.BlockSpec((1, 1, th + 2, Wp + 2, Cin),
                              lambda b, t: (b, t, 0, 0, 0))

    if Cin <= 8 and not pool:
        fatten = "full9"
        wk = w.reshape(9 * Cin, Cout)
        w_spec = pl.BlockSpec((9 * Cin, Cout), lambda b, t: (0, 0))
    else:
        fatten = "ky3"
        wk = jnp.swapaxes(w, 0, 1).reshape(3, 3 * Cin, Cout)
        w_spec = pl.BlockSpec((3, 3 * Cin, Cout), lambda b, t: (0, 0, 0))

    out = pl.pallas_call(
        functools.partial(_conv_kernel, th=th, wp=Wp, w_out=W, pool=pool,
                          fatten=fatten),
        out_shape=jax.ShapeDtypeStruct((B, nt * tho, wo, Cout), jnp.bfloat16),
        grid=(B, nt),
        in_specs=[
            x_spec,
            w_spec,
            pl.BlockSpec((1, Cout), lambda b, t: (0, 0)),
            pl.BlockSpec((1, Cout), lambda b, t: (0, 0)),
        ],
        out_specs=pl.BlockSpec((1, tho, wo, Cout), lambda b, t: (b, t, 0, 0)),
        compiler_params=pltpu.CompilerParams(
            dimension_semantics=("parallel", "parallel"),
            vmem_limit_bytes=_VMEM_LIMIT),
    )(xs, wk, scale.reshape(1, Cout).astype(jnp.float32),
      bias.reshape(1, Cout).astype(jnp.float32))
    return out


def _fc_kernel(a_ref, b_ref, s_ref, c_ref, o_ref, acc_ref, *, relu):
    @pl.when(pl.program_id(1) == 0)
    def _init():
        acc_ref[...] = jnp.zeros_like(acc_ref)

    acc_ref[...] += jnp.dot(a_ref[...], b_ref[...],
                            preferred_element_type=jnp.float32)

    @pl.when(pl.program_id(1) == pl.num_programs(1) - 1)
    def _finalize():
        y = acc_ref[...] * s_ref[...] + c_ref[...]
        if relu:
            y = jnp.maximum(y, 0.0)
        o_ref[...] = y.astype(o_ref.dtype)


def _fc(a, b, scale, bias, *, relu, tn, tk, out_dtype=jnp.bfloat16):
    M, K = a.shape
    K2, N = b.shape
    assert K == K2 and K % tk == 0
    Np = _round_up(N, tn)

    a = a.astype(jnp.bfloat16)
    b = b.astype(jnp.bfloat16)
    if Np != N:
        b = jnp.pad(b, ((0, 0), (0, Np - N)))
        scale = jnp.pad(scale, (0, Np - N))
        bias = jnp.pad(bias, (0, Np - N))
    s = scale.reshape(1, Np).astype(jnp.float32)
    c = bias.reshape(1, Np).astype(jnp.float32)

    grid = (Np // tn, K // tk)
    out = pl.pallas_call(
        functools.partial(_fc_kernel, relu=relu),
        out_shape=jax.ShapeDtypeStruct((M, Np), out_dtype),
        grid=grid,
        in_specs=[
            pl.BlockSpec((M, tk), lambda j, k: (0, k)),
            pl.BlockSpec((tk, tn), lambda j, k: (k, j)),
            pl.BlockSpec((1, tn), lambda j, k: (0, j)),
            pl.BlockSpec((1, tn), lambda j, k: (0, j)),
        ],
        out_specs=pl.BlockSpec((M, tn), lambda j, k: (0, j)),
        scratch_shapes=[pltpu.VMEM((M, tn), jnp.float32)],
        compiler_params=pltpu.CompilerParams(
            dimension_semantics=("parallel", "arbitrary"),
            vmem_limit_bytes=_VMEM_LIMIT),
    )(a, b, s, c)
    if Np != N:
        out = out[:, :N]
    return out


_POOL_AFTER = {1, 3, 6, 9, 12}


def kernel(x_nchw,
           conv0_w, conv0_s, conv0_b,
           conv1_w, conv1_s, conv1_b,
           conv2_w, conv2_s, conv2_b,
           conv3_w, conv3_s, conv3_b,
           conv4_w, conv4_s, conv4_b,
           conv5_w, conv5_s, conv5_b,
           conv6_w, conv6_s, conv6_b,
           conv7_w, conv7_s, conv7_b,
           conv8_w, conv8_s, conv8_b,
           conv9_w, conv9_s, conv9_b,
           conv10_w, conv10_s, conv10_b,
           conv11_w, conv11_s, conv11_b,
           conv12_w, conv12_s, conv12_b,
           fc0_w, fc0_s, fc0_b,
           fc1_w, fc1_s, fc1_b,
           fc2_w, fc2_s, fc2_b):
    convs = [
        (conv0_w, conv0_s, conv0_b), (conv1_w, conv1_s, conv1_b),
        (conv2_w, conv2_s, conv2_b), (conv3_w, conv3_s, conv3_b),
        (conv4_w, conv4_s, conv4_b), (conv5_w, conv5_s, conv5_b),
        (conv6_w, conv6_s, conv6_b), (conv7_w, conv7_s, conv7_b),
        (conv8_w, conv8_s, conv8_b), (conv9_w, conv9_s, conv9_b),
        (conv10_w, conv10_s, conv10_b), (conv11_w, conv11_s, conv11_b),
        (conv12_w, conv12_s, conv12_b),
    ]

    x = jnp.transpose(x_nchw, (0, 2, 3, 1)).astype(jnp.bfloat16)
    for i, (w, s, b) in enumerate(convs):
        x = _conv3x3(x, w, s, b, pool=i in _POOL_AFTER)

    B = x.shape[0]
    x = jnp.transpose(x, (0, 3, 1, 2)).reshape(B, -1)

    x = _fc(x, fc0_w, fc0_s, fc0_b, relu=True, tn=512, tk=3584)
    x = _fc(x, fc1_w, fc1_s, fc1_b, relu=True, tn=512, tk=2048)
    x = _fc(x, fc2_w, fc2_s, fc2_b, relu=False, tn=512, tk=2048,
            out_dtype=jnp.float32)
    return x


# speedup vs baseline: 1.0059x; 1.0059x over previous
import functools

import jax
import jax.numpy as jnp
from jax.experimental import pallas as pl
from jax.experimental.pallas import tpu as pltpu

_VMEM_LIMIT = 48 * 1024 * 1024


def _round_up(x, m):
    return ((x + m - 1) // m) * m


def _conv_kernel(x_ref, w_ref, s_ref, b_ref, o_ref, *, th, wp, w_out, pool,
                 fatten):
    cout = o_ref.shape[-1]
    scale = s_ref[...]
    bias = b_ref[...]

    if not pool:
        cin = x_ref.shape[-1]
        xv = x_ref[0, 0]
        if fatten == "full9":
            xcat = jnp.concatenate(
                [xv[ky:ky + th, kx:kx + wp]
                 for ky in range(3) for kx in range(3)], axis=-1)
            acc = jnp.dot(xcat.reshape(th * wp, 9 * cin), w_ref[...],
                          preferred_element_type=jnp.float32)
        elif fatten == "ky3":
            xcat = jnp.concatenate([xv[ky:ky + th] for ky in range(3)],
                                   axis=-1)
            acc = None
            for kx in range(3):
                lhs = xcat[:, kx:kx + wp].reshape(th * wp, 3 * cin)
                d = jnp.dot(lhs, w_ref[kx],
                            preferred_element_type=jnp.float32)
                acc = d if acc is None else acc + d
        else:
            acc = None
            for ky in range(3):
                xr = xv[ky:ky + th]
                for kx in range(3):
                    lhs = xr[:, kx:kx + wp, :].reshape(th * wp, cin)
                    d = jnp.dot(lhs, w_ref[ky, kx],
                                preferred_element_type=jnp.float32)
                    acc = d if acc is None else acc + d
        y = jnp.maximum(acc * scale + bias, 0.0)
        o_ref[0] = y.reshape(th, wp, cout)[:, :w_out].astype(o_ref.dtype)
        return

    cin = x_ref.shape[-1] // 2
    wh = wp // 2
    wo = w_out // 2
    xf = x_ref[0, 0]
    xe = xf[:, :, :cin]
    xo = xf[:, :, cin:]

    def tap_dots(taps, wref_for_kx):
        acc = None
        for kx, tap in enumerate(taps):
            lhs = tap.reshape(th * wh, tap.shape[-1])
            d = jnp.dot(lhs, wref_for_kx(kx),
                        preferred_element_type=jnp.float32)
            acc = d if acc is None else acc + d
        return acc

    if fatten == "ky3":
        xec = jnp.concatenate([xe[ky:ky + th] for ky in range(3)], axis=-1)
        xoc = jnp.concatenate([xo[ky:ky + th] for ky in range(3)], axis=-1)
        acc_e = tap_dots([xec[:, 0:wh], xoc[:, 0:wh], xec[:, 1:1 + wh]],
                         lambda kx: w_ref[kx])
        acc_o = tap_dots([xoc[:, 0:wh], xec[:, 1:1 + wh], xoc[:, 1:1 + wh]],
                         lambda kx: w_ref[kx])
    else:
        acc_e = acc_o = None
        for ky in range(3):
            xer = xe[ky:ky + th]
            xor = xo[ky:ky + th]
            for kx, tap in enumerate((xer[:, 0:wh], xor[:, 0:wh],
                                      xer[:, 1:1 + wh])):
                d = jnp.dot(tap.reshape(th * wh, cin), w_ref[ky, kx],
                            preferred_element_type=jnp.float32)
                acc_e = d if acc_e is None else acc_e + d
            for kx, tap in enumerate((xor[:, 0:wh], xer[:, 1:1 + wh],
                                      xor[:, 1:1 + wh])):
                d = jnp.dot(tap.reshape(th * wh, cin), w_ref[ky, kx],
                            preferred_element_type=jnp.float32)
                acc_o = d if acc_o is None else acc_o + d

    y_e = jnp.maximum(acc_e * scale + bias, 0.0)
    y_o = jnp.maximum(acc_o * scale + bias, 0.0)
    y = jnp.maximum(y_e, y_o).reshape(th, wh, cout)[:, :wo]
    y = y.reshape(th // 2, 2, wo, cout).max(axis=1)
    o_ref[0] = y.astype(o_ref.dtype)


def _conv3x3(x, w, scale, bias, *, pool):
    B, H, W, Cin = x.shape
    Cout = w.shape[-1]

    if Cin % 8 != 0:
        pc = 8 - Cin % 8
        x = jnp.pad(x, ((0, 0), (0, 0), (0, 0), (0, pc)))
        w = jnp.pad(w, ((0, 0), (0, 0), (0, pc), (0, 0)))
        Cin += pc

    Wp = _round_up(W, 8)
    xp = jnp.pad(x, ((0, 0), (1, 1), (1, Wp - W + 1), (0, 0)))

    th = 56 if H % 56 == 0 else H
    nt = H // th
    if nt > 1:
        xs = jnp.stack([xp[:, t * th:t * th + th + 2] for t in range(nt)],
                       axis=1)
    else:
        xs = xp[:, None]
    tho, wo = (th // 2, W // 2) if pool else (th, W)

    if pool:
        xs = xs.reshape(B, nt, th + 2, (Wp + 2) // 2, 2 * Cin)
        x_spec = pl.BlockSpec((1, 1, th + 2, (Wp + 2) // 2, 2 * Cin),
                              lambda b, t: (b, t, 0, 0, 0))
    else:
        x_spec = pl.BlockSpec((1, 1, th + 2, Wp + 2, Cin),
                              lambda b, t: (b, t, 0, 0, 0))

    if Cin <= 8 and not pool:
        fatten = "full9"
        wk = w.reshape(9 * Cin, Cout)
        w_spec = pl.BlockSpec((9 * Cin, Cout), lambda b, t: (0, 0))
    elif Cin <= 128:
        fatten = "ky3"
        wk = jnp.swapaxes(w, 0, 1).reshape(3, 3 * Cin, Cout)
        w_spec = pl.BlockSpec((3, 3 * Cin, Cout), lambda b, t: (0, 0, 0))
    else:
        fatten = "none"
        wk = w
        w_spec = pl.BlockSpec((3, 3, Cin, Cout), lambda b, t: (0, 0, 0, 0))

    out = pl.pallas_call(
        functools.partial(_conv_kernel, th=th, wp=Wp, w_out=W, pool=pool,
                          fatten=fatten),
        out_shape=jax.ShapeDtypeStruct((B, nt * tho, wo, Cout), jnp.bfloat16),
        grid=(B, nt),
        in_specs=[
            x_spec,
            w_spec,
            pl.BlockSpec((1, Cout), lambda b, t: (0, 0)),
            pl.BlockSpec((1, Cout), lambda b, t: (0, 0)),
        ],
        out_specs=pl.BlockSpec((1, tho, wo, Cout), lambda b, t: (b, t, 0, 0)),
        compiler_params=pltpu.CompilerParams(
            dimension_semantics=("parallel", "parallel"),
            vmem_limit_bytes=_VMEM_LIMIT),
    )(xs, wk, scale.reshape(1, Cout).astype(jnp.float32),
      bias.reshape(1, Cout).astype(jnp.float32))
    return out


def _fc_kernel(a_ref, b_ref, s_ref, c_ref, o_ref, acc_ref, *, relu):
    @pl.when(pl.program_id(1) == 0)
    def _init():
        acc_ref[...] = jnp.zeros_like(acc_ref)

    acc_ref[...] += jnp.dot(a_ref[...], b_ref[...],
                            preferred_element_type=jnp.float32)

    @pl.when(pl.program_id(1) == pl.num_programs(1) - 1)
    def _finalize():
        y = acc_ref[...] * s_ref[...] + c_ref[...]
        if relu:
            y = jnp.maximum(y, 0.0)
        o_ref[...] = y.astype(o_ref.dtype)


def _fc(a, b, scale, bias, *, relu, tn, tk, out_dtype=jnp.bfloat16):
    M, K = a.shape
    K2, N = b.shape
    assert K == K2 and K % tk == 0
    Np = _round_up(N, tn)

    a = a.astype(jnp.bfloat16)
    b = b.astype(jnp.bfloat16)
    if Np != N:
        b = jnp.pad(b, ((0, 0), (0, Np - N)))
        scale = jnp.pad(scale, (0, Np - N))
        bias = jnp.pad(bias, (0, Np - N))
    s = scale.reshape(1, Np).astype(jnp.float32)
    c = bias.reshape(1, Np).astype(jnp.float32)

    grid = (Np // tn, K // tk)
    out = pl.pallas_call(
        functools.partial(_fc_kernel, relu=relu),
        out_shape=jax.ShapeDtypeStruct((M, Np), out_dtype),
        grid=grid,
        in_specs=[
            pl.BlockSpec((M, tk), lambda j, k: (0, k)),
            pl.BlockSpec((tk, tn), lambda j, k: (k, j)),
            pl.BlockSpec((1, tn), lambda j, k: (0, j)),
            pl.BlockSpec((1, tn), lambda j, k: (0, j)),
        ],
        out_specs=pl.BlockSpec((M, tn), lambda j, k: (0, j)),
        scratch_shapes=[pltpu.VMEM((M, tn), jnp.float32)],
        compiler_params=pltpu.CompilerParams(
            dimension_semantics=("parallel", "arbitrary"),
            vmem_limit_bytes=_VMEM_LIMIT),
    )(a, b, s, c)
    if Np != N:
        out = out[:, :N]
    return out


_POOL_AFTER = {1, 3, 6, 9, 12}


def kernel(x_nchw,
           conv0_w, conv0_s, conv0_b,
           conv1_w, conv1_s, conv1_b,
           conv2_w, conv2_s, conv2_b,
           conv3_w, conv3_s, conv3_b,
           conv4_w, conv4_s, conv4_b,
           conv5_w, conv5_s, conv5_b,
           conv6_w, conv6_s, conv6_b,
           conv7_w, conv7_s, conv7_b,
           conv8_w, conv8_s, conv8_b,
           conv9_w, conv9_s, conv9_b,
           conv10_w, conv10_s, conv10_b,
           conv11_w, conv11_s, conv11_b,
           conv12_w, conv12_s, conv12_b,
           fc0_w, fc0_s, fc0_b,
           fc1_w, fc1_s, fc1_b,
           fc2_w, fc2_s, fc2_b):
    convs = [
        (conv0_w, conv0_s, conv0_b), (conv1_w, conv1_s, conv1_b),
        (conv2_w, conv2_s, conv2_b), (conv3_w, conv3_s, conv3_b),
        (conv4_w, conv4_s, conv4_b), (conv5_w, conv5_s, conv5_b),
        (conv6_w, conv6_s, conv6_b), (conv7_w, conv7_s, conv7_b),
        (conv8_w, conv8_s, conv8_b), (conv9_w, conv9_s, conv9_b),
        (conv10_w, conv10_s, conv10_b), (conv11_w, conv11_s, conv11_b),
        (conv12_w, conv12_s, conv12_b),
    ]

    x = jnp.transpose(x_nchw, (0, 2, 3, 1)).astype(jnp.bfloat16)
    for i, (w, s, b) in enumerate(convs):
        x = _conv3x3(x, w, s, b, pool=i in _POOL_AFTER)

    B = x.shape[0]
    x = jnp.transpose(x, (0, 3, 1, 2)).reshape(B, -1)

    x = _fc(x, fc0_w, fc0_s, fc0_b, relu=True, tn=512, tk=3584)
    x = _fc(x, fc1_w, fc1_s, fc1_b, relu=True, tn=512, tk=2048)
    x = _fc(x, fc2_w, fc2_s, fc2_b, relu=False, tn=512, tk=2048,
            out_dtype=jnp.float32)
    return x
